# Initial kernel scaffold; baseline (speedup 1.0000x reference)
import jax
import jax.numpy as jnp
from jax.experimental import pallas as pl


def kernel(fs_s, fs_t, idx, contrast_idx, W_s, b_s, W_t, b_t, memory_v1, memory_v2):
    raise NotImplementedError("write your pallas kernel here")


# speedup vs baseline: 1.2025x; 1.2025x over previous
import dataclasses
import functools

import jax
import jax.numpy as jnp
from jax import lax
from jax.experimental import pallas as pl
from jax.experimental.pallas import tpu as pltpu
from jax.experimental.pallas import tpu_sc as plsc

B = 1024
FEAT = 128
K = 256
KP = K + 1
KPAD = 272
M = B * KPAD
N_DATA = 100000
TEMP = 0.07
MOM = 0.5
EPS = 1e-07
RESIDUAL = K / N_DATA

NC = 2
NS = 16
LANES = 16
NW = NC * NS
SAMP_PW = B // NW
CHUNK = SAMP_PW * KPAD
GW = 128


def _vector_mesh():
    return plsc.VectorSubcoreMesh(core_axis_name="c", subcore_axis_name="s")


def _sc_compiler_params():
    cp = pltpu.CompilerParams()
    if "needs_layout_passes" in pltpu.CompilerParams.__dataclass_fields__:
        cp = dataclasses.replace(cp, needs_layout_passes=False)
    return cp


def _tc_embed(fs_s, fs_t, W_s, b_s, W_t, b_t):
    def body(fss_ref, fst_ref, ws_ref, bs_ref, wt_ref, bt_ref, fs_out, ft_out):
        x = lax.dot_general(fss_ref[...], ws_ref[...], (((1,), (1,)), ((), ())),
                            preferred_element_type=jnp.float32,
                            precision=lax.Precision.HIGHEST)
        x = x + bs_ref[...]
        n = jnp.sqrt(jnp.sum(x * x, axis=1, keepdims=True))
        fs_out[...] = x / jnp.maximum(n, 1e-12)
        y = lax.dot_general(fst_ref[...], wt_ref[...], (((1,), (1,)), ((), ())),
                            preferred_element_type=jnp.float32,
                            precision=lax.Precision.HIGHEST)
        y = y + bt_ref[...]
        m = jnp.sqrt(jnp.sum(y * y, axis=1, keepdims=True))
        ft_out[...] = y / jnp.maximum(m, 1e-12)

    return pl.pallas_call(
        body,
        out_shape=[jax.ShapeDtypeStruct((B, FEAT), jnp.float32),
                   jax.ShapeDtypeStruct((B, FEAT), jnp.float32)],
    )(fs_s, fs_t, W_s, b_s, W_t, b_t)


NB = 32


def _tc_dots(W1, W2, f_s, f_t):
    def body(w1_ref, w2_ref, fs_ref, ft_ref, rt_ref, rs_ref, us_ref, ut_ref):
        w1 = w1_ref[...]
        w2 = w2_ref[...]
        fs = fs_ref[...]
        ft = ft_ref[...]
        rt_ref[...] = jnp.sum(w1 * ft[:, None, :], axis=2)
        rs_ref[...] = jnp.sum(w2 * fs[:, None, :], axis=2)
        pos_s = w1[:, 0, :]
        pos_t = w2[:, 0, :]
        us = pos_s * MOM + fs * (1.0 - MOM)
        us_ref[...] = us / jnp.sqrt(jnp.sum(us * us, axis=1, keepdims=True))
        ut = pos_t * MOM + ft * (1.0 - MOM)
        ut_ref[...] = ut / jnp.sqrt(jnp.sum(ut * ut, axis=1, keepdims=True))

    return pl.pallas_call(
        body,
        grid=(B // NB,),
        in_specs=[
            pl.BlockSpec((NB, KPAD, FEAT), lambda i: (i, 0, 0)),
            pl.BlockSpec((NB, KPAD, FEAT), lambda i: (i, 0, 0)),
            pl.BlockSpec((NB, FEAT), lambda i: (i, 0)),
            pl.BlockSpec((NB, FEAT), lambda i: (i, 0)),
        ],
        out_specs=[
            pl.BlockSpec((NB, KPAD), lambda i: (i, 0)),
            pl.BlockSpec((NB, KPAD), lambda i: (i, 0)),
            pl.BlockSpec((NB, FEAT), lambda i: (i, 0)),
            pl.BlockSpec((NB, FEAT), lambda i: (i, 0)),
        ],
        out_shape=[jax.ShapeDtypeStruct((B, KPAD), jnp.float32),
                   jax.ShapeDtypeStruct((B, KPAD), jnp.float32),
                   jax.ShapeDtypeStruct((B, FEAT), jnp.float32),
                   jax.ShapeDtypeStruct((B, FEAT), jnp.float32)],
    )(W1, W2, f_s, f_t)


def _tc_umm(f_s, f_t, upd_s, upd_t):
    def body(fs_ref, ft_ref, us_ref, ut_ref, utT_ref, usT_ref):
        utT_ref[...] = lax.dot_general(
            ft_ref[...], us_ref[...], (((1,), (1,)), ((), ())),
            preferred_element_type=jnp.float32, precision=lax.Precision.HIGHEST)
        usT_ref[...] = lax.dot_general(
            fs_ref[...], ut_ref[...], (((1,), (1,)), ((), ())),
            preferred_element_type=jnp.float32, precision=lax.Precision.HIGHEST)

    return pl.pallas_call(
        body,
        out_shape=[jax.ShapeDtypeStruct((B, B), jnp.float32),
                   jax.ShapeDtypeStruct((B, B), jnp.float32)],
    )(f_s, f_t, upd_s, upd_t)


def _tc_loss(R_t, R_s, V, Pt, Ps):
    def body(rt_ref, rs_ref, v_ref, pt_ref, ps_ref, out_ref):
        col = lax.broadcasted_iota(jnp.int32, (B, KPAD), 1)
        valid_col = col < KP
        neg_col = (col >= 1) & valid_col
        v = v_ref[...]
        dt = jnp.where(v > 0.5, pt_ref[...], rt_ref[...])
        dsv = jnp.where(v > 0.5, ps_ref[...], rs_ref[...])
        et = jnp.where(valid_col, jnp.exp(dt * (1.0 / TEMP)), 0.0)
        es = jnp.where(valid_col, jnp.exp(dsv * (1.0 / TEMP)), 0.0)
        z_s = jnp.sum(es) * (N_DATA / (B * KP))
        z_t = jnp.sum(et) * (N_DATA / (B * KP))
        o_s = es / z_s
        o_t = et / z_t
        pos_s = o_s[:, 0:1]
        pos_t = o_t[:, 0:1]
        logD1_s = jnp.log(pos_s / (pos_s + RESIDUAL + EPS))
        logD1_t = jnp.log(pos_t / (pos_t + RESIDUAL + EPS))
        lt_s = jnp.log(RESIDUAL / (o_s + RESIDUAL + EPS))
        lt_t = jnp.log(RESIDUAL / (o_t + RESIDUAL + EPS))
        logD0_s = jnp.sum(jnp.where(neg_col, lt_s, 0.0), axis=1, keepdims=True)
        logD0_t = jnp.sum(jnp.where(neg_col, lt_t, 0.0), axis=1, keepdims=True)
        s_loss = -jnp.sum(logD1_s + logD0_s) * (1.0 / B)
        t_loss = -jnp.sum(logD1_t + logD0_t) * (1.0 / B)
        out_ref[...] = jnp.reshape(s_loss + t_loss, (1, 1))

    return pl.pallas_call(
        body,
        out_shape=jax.ShapeDtypeStruct((1, 1), jnp.float32),
    )(R_t, R_s, V, Pt, Ps)


def _sc_gather_rows(table, idx2d):
    @functools.partial(
        pl.kernel,
        out_type=jax.ShapeDtypeStruct((M, FEAT), jnp.float32),
        mesh=_vector_mesh(),
    )
    def k(table_hbm, idx_hbm, out_hbm):
        def body(i_vmem, o_vmem):
            pltpu.sync_copy(table_hbm.at[i_vmem.at[0]], o_vmem)

        pltpu.emit_pipeline(
            body,
            grid=(M // GW,),
            in_specs=[pl.BlockSpec((1, GW), lambda i: (0, i))],
            out_specs=[pl.BlockSpec((GW, FEAT), lambda i: (i, 0))],
            core_axis_name=("c", "s"),
            dimension_semantics=(pltpu.PARALLEL,),
        )(idx_hbm, out_hbm)

    return k(table, idx2d)


def _sc_build_table(idx):
    @functools.partial(
        pl.kernel,
        out_type=jax.ShapeDtypeStruct((N_DATA,), jnp.int32),
        mesh=_vector_mesh(),
        compiler_params=_sc_compiler_params(),
        scratch_types=[pltpu.VMEM((B,), jnp.int32),
                       pltpu.VMEM((N_DATA,), jnp.int32)],
    )
    def k(idx_hbm, tbl_hbm, idx_v, tbl_v):
        wid = lax.axis_index("s") * NC + lax.axis_index("c")

        @pl.when(wid == 0)
        def _():
            pltpu.sync_copy(idx_hbm, idx_v)

            @pl.loop(0, B // LANES)
            def _(bb):
                iv = idx_v[pl.ds(bb * LANES, LANES)]
                vals = lax.iota(jnp.int32, LANES) + bb * LANES
                plsc.store_scatter(tbl_v, [iv], vals)

            pltpu.sync_copy(tbl_v, tbl_hbm)

    return k(idx)


def _sc_patch(table, idx_full_flat, idx, utT, usT):
    @functools.partial(
        pl.kernel,
        out_type=(jax.ShapeDtypeStruct((M,), jnp.float32),
                  jax.ShapeDtypeStruct((M,), jnp.float32),
                  jax.ShapeDtypeStruct((M,), jnp.float32)),
        mesh=_vector_mesh(),
        compiler_params=_sc_compiler_params(),
        scratch_types=[
            pltpu.VMEM((B,), jnp.int32),
            pltpu.VMEM((CHUNK,), jnp.int32),
            pltpu.VMEM((CHUNK,), jnp.int32),
            pltpu.VMEM((SAMP_PW, B), jnp.float32),
            pltpu.VMEM((SAMP_PW, B), jnp.float32),
            pltpu.VMEM((CHUNK,), jnp.float32),
            pltpu.VMEM((CHUNK,), jnp.float32),
            pltpu.VMEM((CHUNK,), jnp.float32),
            pltpu.SemaphoreType.DMA,
        ],
    )
    def k(tbl_hbm, ifull_hbm, idx_hbm, utT_hbm, usT_hbm,
          v_hbm, pt_hbm, ps_hbm,
          idxall_v, ifull_v, cand_v, urt_v, urs_v, v_v, pt_v, ps_v, sem):
        wid = lax.axis_index("s") * NC + lax.axis_index("c")
        b0 = wid * SAMP_PW
        off0 = wid * CHUNK
        pltpu.sync_copy(idx_hbm, idxall_v)
        pltpu.sync_copy(ifull_hbm.at[pl.ds(off0, CHUNK)], ifull_v)
        pltpu.sync_copy(utT_hbm.at[pl.ds(b0, SAMP_PW)], urt_v)
        pltpu.sync_copy(usT_hbm.at[pl.ds(b0, SAMP_PW)], urs_v)

        @pl.loop(0, CHUNK // GW)
        def _(kk):
            sl = pl.ds(kk * GW, GW)
            pltpu.async_copy(tbl_hbm.at[ifull_v.at[sl]], cand_v.at[sl], sem)

        pltpu.make_async_copy(tbl_hbm.at[pl.ds(0, CHUNK)], cand_v, sem).wait()

        one = jnp.float32(1.0)
        zero = jnp.float32(0.0)

        @pl.loop(0, SAMP_PW)
        def _(t):
            rows = jnp.full((LANES,), t, jnp.int32)

            @pl.loop(0, KPAD // LANES)
            def _(g):
                off = t * KPAD + g * LANES
                c = cand_v[pl.ds(off, LANES)]
                cc = jnp.clip(c, 0, B - 1)
                iv = plsc.load_gather(idxall_v, [cc])
                jf = ifull_v[pl.ds(off, LANES)]
                valid = (iv == jf) & (c >= 0) & (c < B)
                ptv = plsc.load_gather(urt_v, [rows, cc])
                psv = plsc.load_gather(urs_v, [rows, cc])
                v_v[pl.ds(off, LANES)] = jnp.where(valid, one, zero)
                pt_v[pl.ds(off, LANES)] = ptv
                ps_v[pl.ds(off, LANES)] = psv

        pltpu.sync_copy(v_v, v_hbm.at[pl.ds(off0, CHUNK)])
        pltpu.sync_copy(pt_v, pt_hbm.at[pl.ds(off0, CHUNK)])
        pltpu.sync_copy(ps_v, ps_hbm.at[pl.ds(off0, CHUNK)])

    return k(table, idx_full_flat, idx, utT, usT)


def kernel(fs_s, fs_t, idx, contrast_idx, W_s, b_s, W_t, b_t, memory_v1, memory_v2):
    idx = idx.astype(jnp.int32)
    contrast_idx = contrast_idx.astype(jnp.int32)

    f_s, f_t = _tc_embed(fs_s, fs_t, W_s, b_s.reshape(1, FEAT),
                         W_t, b_t.reshape(1, FEAT))

    pad = jnp.zeros((B, KPAD - KP), jnp.int32)
    idx_full = jnp.concatenate([idx[:, None], contrast_idx, pad], axis=1)

    W1 = _sc_gather_rows(memory_v1, idx_full.reshape(1, M))
    W2 = _sc_gather_rows(memory_v2, idx_full.reshape(1, M))

    R_t, R_s, upd_s, upd_t = _tc_dots(W1.reshape(B, KPAD, FEAT),
                                      W2.reshape(B, KPAD, FEAT), f_s, f_t)
    U_tT, U_sT = _tc_umm(f_s, f_t, upd_s, upd_t)

    table = _sc_build_table(idx)
    V, Pt, Ps = _sc_patch(table, idx_full.reshape(M), idx, U_tT, U_sT)

    loss = _tc_loss(R_t, R_s, V.reshape(B, KPAD), Pt.reshape(B, KPAD),
                    Ps.reshape(B, KPAD))
    return loss.reshape(())
